# Initial kernel scaffold; baseline (speedup 1.0000x reference)
import jax
import jax.numpy as jnp
from jax.experimental import pallas as pl


def kernel(I, dx_t, dy_t, dz_t):
    raise NotImplementedError("write your pallas kernel here")


# speedup vs baseline: 1.3709x; 1.3709x over previous
import functools

import jax
import jax.numpy as jnp
from jax import lax
from jax.experimental import pallas as pl
from jax.experimental.pallas import tpu as pltpu
from jax.experimental.pallas import tpu_sc as plsc

H = W = D = 128
HP = WP = DP = 130
NVOX = H * W * D
NW = 32
PER_W = NVOX // NW
CH = 4096
NCHUNK = PER_W // CH
VREGS = CH // 16


def _floor_i32(x):
    t = x.astype(jnp.int32)
    tf = t.astype(jnp.float32)
    return jnp.where(tf > x, t - 1, t)


def _warp_body(t0_hbm, t1_hbm, dx_hbm, dy_hbm, dz_hbm, o0_hbm, o1_hbm,
               dxv, dyv, dzv,
               i0, i1, i2, i3, i4, i5, i6, i7,
               a0, a1, a2, a3, a4, a5, a6, a7,
               b0, b1, b2, b3, b4, b5, b6, b7,
               ov0, ov1, sem):
    wid = lax.axis_index("s") * 2 + lax.axis_index("c")
    base = wid * PER_W
    iota = lax.iota(jnp.int32, 16)
    idx_refs = (i0, i1, i2, i3, i4, i5, i6, i7)
    g0_refs = (a0, a1, a2, a3, a4, a5, a6, a7)
    g1_refs = (b0, b1, b2, b3, b4, b5, b6, b7)

    def chunk_body(cc, carry):
        cb = base + cc * CH
        pltpu.sync_copy(dx_hbm.at[pl.ds(cb, CH)], dxv)
        pltpu.sync_copy(dy_hbm.at[pl.ds(cb, CH)], dyv)
        pltpu.sync_copy(dz_hbm.at[pl.ds(cb, CH)], dzv)

        def pass_a(i, c):
            s = pl.ds(i * 16, 16)
            n = (cb + i * 16) + iota
            jj = lax.shift_right_logical(n, 7) & 127
            ii = lax.shift_right_logical(n, 14)
            kk = n & 127
            x = (dxv[s] + jj.astype(jnp.float32)) + 1.0
            y = (dyv[s] + ii.astype(jnp.float32)) + 1.0
            z = (dzv[s] + kk.astype(jnp.float32)) + 1.0
            x0 = _floor_i32(x)
            y0 = _floor_i32(y)
            z0 = _floor_i32(z)
            x0c = jnp.minimum(jnp.maximum(x0, 0), WP - 1)
            x1c = jnp.minimum(jnp.maximum(x0 + 1, 0), WP - 1)
            y0c = jnp.minimum(jnp.maximum(y0, 0), HP - 1)
            y1c = jnp.minimum(jnp.maximum(y0 + 1, 0), HP - 1)
            z0c = jnp.minimum(jnp.maximum(z0, 0), DP - 1)
            z1c = jnp.minimum(jnp.maximum(z0 + 1, 0), DP - 1)
            dxv[s] = x1c.astype(jnp.float32) - x
            dyv[s] = y1c.astype(jnp.float32) - y
            dzv[s] = z1c.astype(jnp.float32) - z
            ya = y0c * (WP * DP)
            yb = y1c * (WP * DP)
            xa = x0c * DP
            xb = x1c * DP
            dz01 = z1c - z0c
            ia = ya + xa + z0c
            ib = yb + xa + z0c
            ic = ya + xb + z0c
            id_ = yb + xb + z0c
            i0[s] = ia
            i1[s] = ib
            i2[s] = ic
            i3[s] = id_
            i4[s] = ia + dz01
            i5[s] = ib + dz01
            i6[s] = ic + dz01
            i7[s] = id_ + dz01
            return c

        lax.fori_loop(0, VREGS, pass_a, 0)

        descs = []
        for r in range(8):
            descs.append(pltpu.async_copy(t0_hbm.at[idx_refs[r]], g0_refs[r], sem))
            descs.append(pltpu.async_copy(t1_hbm.at[idx_refs[r]], g1_refs[r], sem))
        for d in descs:
            d.wait()

        def pass_b(i, c):
            s = pl.ds(i * 16, 16)
            dxw = dxv[s]
            dyw = dyv[s]
            dzw = dzv[s]
            exw = 1.0 - dxw
            eyw = 1.0 - dyw
            ezw = 1.0 - dzw
            zx = dzw * dxw
            zX = dzw * exw
            Zx = ezw * dxw
            ZX = ezw * exw
            wa = zx * dyw
            wb = zx * eyw
            wc = zX * dyw
            wd = zX * eyw
            we = Zx * dyw
            wf = Zx * eyw
            wg = ZX * dyw
            wh = ZX * eyw
            o0 = (((((((wa * a0[s] + wb * a1[s]) + wc * a2[s]) + wd * a3[s])
                     + we * a4[s]) + wf * a5[s]) + wg * a6[s]) + wh * a7[s])
            o1 = (((((((wa * b0[s] + wb * b1[s]) + wc * b2[s]) + wd * b3[s])
                     + we * b4[s]) + wf * b5[s]) + wg * b6[s]) + wh * b7[s])
            ov0[s] = o0
            ov1[s] = o1
            return c

        lax.fori_loop(0, VREGS, pass_b, 0)

        pltpu.sync_copy(ov0, o0_hbm.at[pl.ds(cb, CH)])
        pltpu.sync_copy(ov1, o1_hbm.at[pl.ds(cb, CH)])
        return carry

    lax.fori_loop(0, NCHUNK, chunk_body, 0)


@jax.jit
def _warp(t0, t1, dxf, dyf, dzf):
    mesh = plsc.VectorSubcoreMesh(core_axis_name="c", subcore_axis_name="s")
    f32 = jnp.float32
    i32 = jnp.int32
    scratch = ([pltpu.VMEM((CH,), f32) for _ in range(3)]
               + [pltpu.VMEM((CH,), i32) for _ in range(8)]
               + [pltpu.VMEM((CH,), f32) for _ in range(16)]
               + [pltpu.VMEM((CH,), f32) for _ in range(2)]
               + [pltpu.SemaphoreType.DMA])
    run = functools.partial(
        pl.kernel,
        mesh=mesh,
        out_type=[jax.ShapeDtypeStruct((NVOX,), f32),
                  jax.ShapeDtypeStruct((NVOX,), f32)],
        scratch_types=scratch,
    )(_warp_body)
    return run(t0, t1, dxf, dyf, dzf)


def kernel(I, dx_t, dy_t, dz_t):
    I_pad = jnp.pad(I, ((0, 0), (0, 0), (1, 1), (1, 1), (1, 1)))
    t0 = I_pad[0, 0].reshape(-1)
    t1 = I_pad[0, 1].reshape(-1)
    o0, o1 = _warp(t0, t1, dx_t.reshape(-1), dy_t.reshape(-1), dz_t.reshape(-1))
    return jnp.stack([o0, o1]).reshape(1, 2, H, W, D)
